# baseline (speedup 1.0000x reference)
import functools

import jax
import jax.numpy as jnp
from jax.experimental import pallas as pl
from jax.experimental.pallas import tpu as pltpu
from jax.experimental.pallas import tpu_sc as plsc

N = 576
C = 768
NH = 12
HD = 64
NREG = 64
RROW = 24
TK = 4
MLP_H = 3072
NEG = -1e30


def _layer_norm_f32(x, g, b):
    m = jnp.mean(x, axis=-1, keepdims=True)
    v = jnp.mean((x - m) ** 2, axis=-1, keepdims=True)
    return (x - m) * jax.lax.rsqrt(v + 1e-5) * g + b


def _region_onehot(shape_rn):
    r_idx = jax.lax.broadcasted_iota(jnp.int32, shape_rn, 0)
    n_idx = jax.lax.broadcasted_iota(jnp.int32, shape_rn, 1)
    rid = (n_idx // 72) * 8 + (n_idx % RROW) // 3
    return jnp.where(rid == r_idx, 1.0, 0.0).astype(jnp.float32)


def _qkv_kernel(x_ref, g_ref, b_ref, wt_ref, wb_ref,
                q_ref, k_ref, v_ref, m_ref):
    x = x_ref[0]
    g = g_ref[0]
    b = b_ref[0]
    x1 = _layer_norm_f32(x, g, b)

    wt_bf = wt_ref[...].astype(jnp.bfloat16)
    qkv = jnp.dot(x1.astype(jnp.bfloat16), wt_bf,
                  preferred_element_type=jnp.float32) + wb_ref[0]
    q_ref[0] = qkv[:, :C].astype(jnp.bfloat16)
    k_ref[0] = qkv[:, C:2 * C].astype(jnp.bfloat16)
    v_ref[0] = qkv[:, 2 * C:].astype(jnp.bfloat16)

    hi = jax.lax.Precision.HIGHEST
    pool = _region_onehot((NREG, N)) * (1.0 / 9.0)
    xp = jnp.dot(pool, x1, precision=hi, preferred_element_type=jnp.float32)
    wt = wt_ref[...]
    q_r = jnp.dot(xp, wt[:, :C], precision=hi,
                  preferred_element_type=jnp.float32) + wb_ref[0, :C]
    k_r = jnp.dot(xp, wt[:, C:2 * C], precision=hi,
                  preferred_element_type=jnp.float32) + wb_ref[0, C:2 * C]
    a = jax.lax.dot_general(q_r, k_r, (((1,), (1,)), ((), ())),
                            precision=hi, preferred_element_type=jnp.float32)
    m_ref[0] = a


_NW = 32
_ROWS = 8 * NREG
_RPW = _ROWS // _NW


def _route_sc_kernel(a_hbm, m_hbm, a_v, m_v):
    c = jax.lax.axis_index("c")
    s = jax.lax.axis_index("s")
    wid = s * 2 + c
    base = wid * _RPW
    pltpu.sync_copy(a_hbm.at[pl.ds(base, _RPW)], a_v)
    lane = jax.lax.iota(jnp.int32, 16)
    for r in range(_RPW):
        work = [a_v[r, 16 * j:16 * (j + 1)] for j in range(4)]
        sel = [jnp.zeros((16,), jnp.bool_) for _ in range(4)]
        for _ in range(TK):
            mx = jnp.max(jnp.maximum(jnp.maximum(work[0], work[1]),
                                     jnp.maximum(work[2], work[3])))
            pos = [jnp.min(jnp.where(work[j] == mx, lane + 16 * j, NREG))
                   for j in range(4)]
            first = jnp.minimum(jnp.minimum(pos[0], pos[1]),
                                jnp.minimum(pos[2], pos[3]))
            for j in range(4):
                hit = (lane + 16 * j) == first
                sel[j] = jnp.logical_or(sel[j], hit)
                work[j] = jnp.where(hit, -jnp.inf, work[j])
        for j in range(4):
            m_v[r, 16 * j:16 * (j + 1)] = jnp.where(sel[j], 0.0, NEG)
    pltpu.sync_copy(m_v, m_hbm.at[pl.ds(base, _RPW)])


@functools.partial(
    pl.kernel,
    mesh=plsc.VectorSubcoreMesh(core_axis_name="c", subcore_axis_name="s"),
    out_type=jax.ShapeDtypeStruct((_ROWS, NREG), jnp.float32),
    scratch_types=[
        pltpu.VMEM((_RPW, NREG), jnp.float32),
        pltpu.VMEM((_RPW, NREG), jnp.float32),
    ],
    compiler_params=pltpu.CompilerParams(needs_layout_passes=False),
)
def _route_sc(a_hbm, m_hbm, a_v, m_v):
    _route_sc_kernel(a_hbm, m_hbm, a_v, m_v)


def _block_kernel(x_ref, q_ref, k_ref, v_ref, m_ref, lw_ref, lb_ref,
                  ot_ref, ob_ref, g2_ref, b2_ref, f1t_ref, f1b_ref,
                  f2t_ref, f2b_ref, y_ref):
    x = x_ref[0]
    q = q_ref[0]
    k = k_ref[0]
    v = v_ref[0]

    e_rn = _region_onehot((NREG, N))
    m64 = m_ref[0]
    inner = jnp.dot(m64, e_rn, preferred_element_type=jnp.float32)
    mask = jax.lax.dot_general(e_rn, inner, (((0,), (0,)), ((), ())),
                               preferred_element_type=jnp.float32)

    scale = float(C) ** (-0.5)
    qs = (q.astype(jnp.float32) * scale).astype(jnp.bfloat16)
    heads = []
    for h in range(NH):
        sl = slice(h * HD, (h + 1) * HD)
        s = jax.lax.dot_general(qs[:, sl], k[:, sl], (((1,), (1,)), ((), ())),
                                preferred_element_type=jnp.float32)
        s = s + mask
        mx = jnp.max(s, axis=1, keepdims=True)
        e = jnp.exp(s - mx)
        p = e / jnp.sum(e, axis=1, keepdims=True)
        heads.append(jnp.dot(p.astype(jnp.bfloat16), v[:, sl],
                             preferred_element_type=jnp.float32))
    attn = jnp.concatenate(heads, axis=1)

    vf = v.astype(jnp.float32)
    wcol = jax.lax.broadcasted_iota(jnp.int32, (N, 1), 0) % RROW
    acc = jnp.zeros((N, C), jnp.float32)
    for kh in range(3):
        for kw in range(3):
            s = RROW * (kh - 1) + (kw - 1)
            if s > 0:
                sh = jnp.concatenate(
                    [vf[s:], jnp.zeros((s, C), jnp.float32)], axis=0)
            elif s < 0:
                sh = jnp.concatenate(
                    [jnp.zeros((-s, C), jnp.float32), vf[:N + s]], axis=0)
            else:
                sh = vf
            if kw == 0:
                sh = jnp.where(wcol >= 1, sh, 0.0)
            elif kw == 2:
                sh = jnp.where(wcol <= RROW - 2, sh, 0.0)
            acc = acc + sh * lw_ref[kh * 3 + kw][None, :]
    lepe = acc + lb_ref[0]

    ab = (attn + lepe).astype(jnp.bfloat16)
    proj = jnp.dot(ab, ot_ref[...], preferred_element_type=jnp.float32)
    xm = x + proj + ob_ref[0]

    x2 = _layer_norm_f32(xm, g2_ref[0], b2_ref[0]).astype(jnp.bfloat16)
    yacc = jnp.zeros((N, C), jnp.float32)
    chunk = MLP_H // 4
    for j in range(4):
        sl = slice(j * chunk, (j + 1) * chunk)
        h1 = jnp.dot(x2, f1t_ref[:, sl],
                     preferred_element_type=jnp.float32) + f1b_ref[0, sl]
        gl = 0.5 * h1 * (1.0 + jax.lax.erf(h1 * (2.0 ** -0.5)))
        yacc = yacc + jnp.dot(gl.astype(jnp.bfloat16), f2t_ref[sl, :],
                              preferred_element_type=jnp.float32)
    y_ref[0] = xm + yacc + f2b_ref[0]


def _full(shape):
    return pl.BlockSpec(shape, lambda b: (0,) * len(shape))


def _batched(shape):
    return pl.BlockSpec((1,) + shape, lambda b: (b,) + (0,) * len(shape))


@jax.jit
def kernel(x, norm1_g, norm1_b, qkv_w, qkv_b, lepe_w, lepe_b, out_w, out_b,
           norm2_g, norm2_b, fc1_w, fc1_b, fc2_w, fc2_b):
    B = x.shape[0]
    f32 = jnp.float32
    bf16 = jnp.bfloat16

    q, k, v, a_r = pl.pallas_call(
        _qkv_kernel,
        grid=(B,),
        in_specs=[
            _batched((N, C)),
            _full((1, C)), _full((1, C)),
            _full((C, 3 * C)), _full((1, 3 * C)),
        ],
        out_specs=[
            _batched((N, C)), _batched((N, C)), _batched((N, C)),
            _batched((NREG, NREG)),
        ],
        out_shape=[
            jax.ShapeDtypeStruct((B, N, C), bf16),
            jax.ShapeDtypeStruct((B, N, C), bf16),
            jax.ShapeDtypeStruct((B, N, C), bf16),
            jax.ShapeDtypeStruct((B, NREG, NREG), f32),
        ],
    )(x, norm1_g.reshape(1, C), norm1_b.reshape(1, C),
      qkv_w.T, qkv_b.reshape(1, 3 * C))

    mask64 = _route_sc(a_r.reshape(_ROWS, NREG)).reshape(B, NREG, NREG)

    lw9 = jnp.transpose(lepe_w, (1, 2, 3, 0)).reshape(9, C)
    y = pl.pallas_call(
        _block_kernel,
        grid=(B,),
        in_specs=[
            _batched((N, C)), _batched((N, C)), _batched((N, C)),
            _batched((N, C)), _batched((NREG, NREG)),
            _full((9, C)), _full((1, C)),
            _full((C, C)), _full((1, C)),
            _full((1, C)), _full((1, C)),
            _full((C, MLP_H)), _full((1, MLP_H)),
            _full((MLP_H, C)), _full((1, C)),
        ],
        out_specs=_batched((N, C)),
        out_shape=jax.ShapeDtypeStruct((B, N, C), f32),
    )(x, q, k, v, mask64,
      lw9, lepe_b.reshape(1, C),
      out_w.T.astype(bf16), out_b.reshape(1, C),
      norm2_g.reshape(1, C), norm2_b.reshape(1, C),
      fc1_w.T.astype(bf16), fc1_b.reshape(1, MLP_H),
      fc2_w.T.astype(bf16), fc2_b.reshape(1, C))
    return y


# speedup vs baseline: 1.0219x; 1.0219x over previous
import functools

import jax
import jax.numpy as jnp
from jax.experimental import pallas as pl
from jax.experimental.pallas import tpu as pltpu
from jax.experimental.pallas import tpu_sc as plsc

N = 576
C = 768
NH = 12
HD = 64
NREG = 64
RROW = 24
TK = 4
MLP_H = 3072
NEG = -1e30


def _layer_norm_f32(x, g, b):
    m = jnp.mean(x, axis=-1, keepdims=True)
    v = jnp.mean((x - m) ** 2, axis=-1, keepdims=True)
    return (x - m) * jax.lax.rsqrt(v + 1e-5) * g + b


def _region_onehot(shape_rn):
    r_idx = jax.lax.broadcasted_iota(jnp.int32, shape_rn, 0)
    n_idx = jax.lax.broadcasted_iota(jnp.int32, shape_rn, 1)
    rid = (n_idx // 72) * 8 + (n_idx % RROW) // 3
    return jnp.where(rid == r_idx, 1.0, 0.0).astype(jnp.float32)


def _qkv_kernel(x_ref, g_ref, b_ref, wt_ref, wb_ref,
                q_ref, k_ref, v_ref, m_ref):
    x = x_ref[0]
    g = g_ref[0]
    b = b_ref[0]
    x1 = _layer_norm_f32(x, g, b)

    wt_bf = wt_ref[...].astype(jnp.bfloat16)
    qkv = jnp.dot(x1.astype(jnp.bfloat16), wt_bf,
                  preferred_element_type=jnp.float32) + wb_ref[0]
    q_ref[0] = (qkv[:, :C] * (float(C) ** -0.5)).astype(jnp.bfloat16)
    k_ref[0] = qkv[:, C:2 * C].astype(jnp.bfloat16)
    v_ref[0] = qkv[:, 2 * C:].astype(jnp.bfloat16)

    hi = jax.lax.Precision.HIGHEST
    pool = _region_onehot((NREG, N)) * (1.0 / 9.0)
    xp = jnp.dot(pool, x1, precision=hi, preferred_element_type=jnp.float32)
    wt = wt_ref[...]
    q_r = jnp.dot(xp, wt[:, :C], precision=hi,
                  preferred_element_type=jnp.float32) + wb_ref[0, :C]
    k_r = jnp.dot(xp, wt[:, C:2 * C], precision=hi,
                  preferred_element_type=jnp.float32) + wb_ref[0, C:2 * C]
    a = jax.lax.dot_general(q_r, k_r, (((1,), (1,)), ((), ())),
                            precision=hi, preferred_element_type=jnp.float32)
    m_ref[0] = a


_NW = 32
_ROWS = 8 * NREG
_RPW = _ROWS // _NW


def _route_sc_kernel(a_hbm, m_hbm, a_v, m_v):
    c = jax.lax.axis_index("c")
    s = jax.lax.axis_index("s")
    wid = s * 2 + c
    base = wid * _RPW
    pltpu.sync_copy(a_hbm.at[pl.ds(base, _RPW)], a_v)
    lane = jax.lax.iota(jnp.int32, 16)
    for r in range(_RPW):
        work = [a_v[r, 16 * j:16 * (j + 1)] for j in range(4)]
        sel = [jnp.zeros((16,), jnp.bool_) for _ in range(4)]
        for _ in range(TK):
            mx = jnp.max(jnp.maximum(jnp.maximum(work[0], work[1]),
                                     jnp.maximum(work[2], work[3])))
            pos = [jnp.min(jnp.where(work[j] == mx, lane + 16 * j, NREG))
                   for j in range(4)]
            first = jnp.minimum(jnp.minimum(pos[0], pos[1]),
                                jnp.minimum(pos[2], pos[3]))
            for j in range(4):
                hit = (lane + 16 * j) == first
                sel[j] = jnp.logical_or(sel[j], hit)
                work[j] = jnp.where(hit, -jnp.inf, work[j])
        for j in range(4):
            m_v[r, 16 * j:16 * (j + 1)] = jnp.where(sel[j], 0.0, NEG)
    pltpu.sync_copy(m_v, m_hbm.at[pl.ds(base, _RPW)])


@functools.partial(
    pl.kernel,
    mesh=plsc.VectorSubcoreMesh(core_axis_name="c", subcore_axis_name="s"),
    out_type=jax.ShapeDtypeStruct((_ROWS, NREG), jnp.float32),
    scratch_types=[
        pltpu.VMEM((_RPW, NREG), jnp.float32),
        pltpu.VMEM((_RPW, NREG), jnp.float32),
    ],
    compiler_params=pltpu.CompilerParams(needs_layout_passes=False),
)
def _route_sc(a_hbm, m_hbm, a_v, m_v):
    _route_sc_kernel(a_hbm, m_hbm, a_v, m_v)


def _block_kernel(x_ref, q_ref, k_ref, v_ref, m_ref, lw_ref, lb_ref,
                  ot_ref, ob_ref, g2_ref, b2_ref, f1t_ref, f1b_ref,
                  f2t_ref, f2b_ref, y_ref):
    x = x_ref[0]
    q = q_ref[0]
    k = k_ref[0]
    v = v_ref[0]

    e_rn = _region_onehot((NREG, N))
    m64 = m_ref[0]
    inner = jnp.dot(m64, e_rn, preferred_element_type=jnp.float32)
    mask = jax.lax.dot_general(e_rn, inner, (((0,), (0,)), ((), ())),
                               preferred_element_type=jnp.float32)

    wcol = jax.lax.broadcasted_iota(jnp.int32, (N, 1), 0) % RROW
    zrow = jnp.zeros((RROW + 1, C), jnp.bfloat16)
    lepe = jnp.zeros((N, C), jnp.float32) + lb_ref[0]
    for kh in range(3):
        for kw in range(3):
            s = RROW * (kh - 1) + (kw - 1)
            if s > 0:
                sh = jnp.concatenate([v[s:], zrow[:s]], axis=0)
            elif s < 0:
                sh = jnp.concatenate([zrow[:-s], v[:N + s]], axis=0)
            else:
                sh = v
            if kw == 0:
                sh = jnp.where(wcol >= 1, sh, jnp.bfloat16(0))
            elif kw == 2:
                sh = jnp.where(wcol <= RROW - 2, sh, jnp.bfloat16(0))
            lepe = lepe + sh * lw_ref[kh * 3 + kw][None, :].astype(jnp.bfloat16)

    ones_bf = jnp.ones((N, 128), jnp.bfloat16)
    mask_bf = mask.astype(jnp.bfloat16)
    heads = []
    for h in range(NH):
        sl = slice(h * HD, (h + 1) * HD)
        s = jax.lax.dot_general(q[:, sl], k[:, sl], (((1,), (1,)), ((), ())),
                                preferred_element_type=jnp.float32)
        e = jnp.exp(s.astype(jnp.bfloat16) + mask_bf)
        denom = jnp.dot(e, ones_bf, preferred_element_type=jnp.float32)
        o = jnp.dot(e, v[:, sl], preferred_element_type=jnp.float32)
        heads.append(o / denom[:, :HD])

    proj = jnp.zeros((N, C), jnp.float32)
    for hp in range(NH // 2):
        sl = slice(hp * 128, (hp + 1) * 128)
        ab = (jnp.concatenate([heads[2 * hp], heads[2 * hp + 1]], axis=1)
              + lepe[:, sl]).astype(jnp.bfloat16)
        proj = proj + jnp.dot(ab, ot_ref[sl, :],
                              preferred_element_type=jnp.float32)
    xm = x + proj + ob_ref[0]

    x2 = _layer_norm_f32(xm, g2_ref[0], b2_ref[0]).astype(jnp.bfloat16)
    yacc = jnp.zeros((N, C), jnp.float32)
    chunk = MLP_H // 4
    for j in range(4):
        sl = slice(j * chunk, (j + 1) * chunk)
        h1 = jnp.dot(x2, f1t_ref[:, sl],
                     preferred_element_type=jnp.float32) + f1b_ref[0, sl]
        gl = 0.5 * h1 * (1.0 + jax.lax.erf(h1 * (2.0 ** -0.5)))
        yacc = yacc + jnp.dot(gl.astype(jnp.bfloat16), f2t_ref[sl, :],
                              preferred_element_type=jnp.float32)
    y_ref[0] = xm + yacc + f2b_ref[0]


def _full(shape):
    return pl.BlockSpec(shape, lambda b: (0,) * len(shape))


def _batched(shape):
    return pl.BlockSpec((1,) + shape, lambda b: (b,) + (0,) * len(shape))


@jax.jit
def kernel(x, norm1_g, norm1_b, qkv_w, qkv_b, lepe_w, lepe_b, out_w, out_b,
           norm2_g, norm2_b, fc1_w, fc1_b, fc2_w, fc2_b):
    B = x.shape[0]
    f32 = jnp.float32
    bf16 = jnp.bfloat16

    q, k, v, a_r = pl.pallas_call(
        _qkv_kernel,
        grid=(B,),
        in_specs=[
            _batched((N, C)),
            _full((1, C)), _full((1, C)),
            _full((C, 3 * C)), _full((1, 3 * C)),
        ],
        out_specs=[
            _batched((N, C)), _batched((N, C)), _batched((N, C)),
            _batched((NREG, NREG)),
        ],
        out_shape=[
            jax.ShapeDtypeStruct((B, N, C), bf16),
            jax.ShapeDtypeStruct((B, N, C), bf16),
            jax.ShapeDtypeStruct((B, N, C), bf16),
            jax.ShapeDtypeStruct((B, NREG, NREG), f32),
        ],
    )(x, norm1_g.reshape(1, C), norm1_b.reshape(1, C),
      qkv_w.T, qkv_b.reshape(1, 3 * C))

    mask64 = _route_sc(a_r.reshape(_ROWS, NREG)).reshape(B, NREG, NREG)

    lw9 = jnp.transpose(lepe_w, (1, 2, 3, 0)).reshape(9, C)
    y = pl.pallas_call(
        _block_kernel,
        grid=(B,),
        in_specs=[
            _batched((N, C)), _batched((N, C)), _batched((N, C)),
            _batched((N, C)), _batched((NREG, NREG)),
            _full((9, C)), _full((1, C)),
            _full((C, C)), _full((1, C)),
            _full((1, C)), _full((1, C)),
            _full((C, MLP_H)), _full((1, MLP_H)),
            _full((MLP_H, C)), _full((1, C)),
        ],
        out_specs=_batched((N, C)),
        out_shape=jax.ShapeDtypeStruct((B, N, C), f32),
    )(x, q, k, v, mask64,
      lw9, lepe_b.reshape(1, C),
      out_w.T.astype(bf16), out_b.reshape(1, C),
      norm2_g.reshape(1, C), norm2_b.reshape(1, C),
      fc1_w.T.astype(bf16), fc1_b.reshape(1, MLP_H),
      fc2_w.T.astype(bf16), fc2_b.reshape(1, C))
    return y


# speedup vs baseline: 1.0435x; 1.0212x over previous
import functools

import jax
import jax.numpy as jnp
from jax.experimental import pallas as pl
from jax.experimental.pallas import tpu as pltpu
from jax.experimental.pallas import tpu_sc as plsc

N = 576
C = 768
NH = 12
HD = 64
NREG = 64
RROW = 24
TK = 4
MLP_H = 3072
NEG = -1e30


def _layer_norm_f32(x, g, b):
    m = jnp.mean(x, axis=-1, keepdims=True)
    v = jnp.mean((x - m) ** 2, axis=-1, keepdims=True)
    return (x - m) * jax.lax.rsqrt(v + 1e-5) * g + b


def _region_onehot(shape_rn):
    r_idx = jax.lax.broadcasted_iota(jnp.int32, shape_rn, 0)
    n_idx = jax.lax.broadcasted_iota(jnp.int32, shape_rn, 1)
    rid = (n_idx // 72) * 8 + (n_idx % RROW) // 3
    return jnp.where(rid == r_idx, 1.0, 0.0).astype(jnp.float32)


def _qkv_kernel(x_ref, g_ref, b_ref, wt_ref, wb_ref,
                q_ref, k_ref, v_ref, m_ref):
    x = x_ref[0]
    g = g_ref[0]
    b = b_ref[0]
    x1 = _layer_norm_f32(x, g, b)

    wt_bf = wt_ref[...].astype(jnp.bfloat16)
    qkv = jnp.dot(x1.astype(jnp.bfloat16), wt_bf,
                  preferred_element_type=jnp.float32) + wb_ref[0]
    q_ref[0] = (qkv[:, :C] * (float(C) ** -0.5)).astype(jnp.bfloat16)
    k_ref[0] = qkv[:, C:2 * C].astype(jnp.bfloat16)
    v_ref[0] = qkv[:, 2 * C:].astype(jnp.bfloat16)

    hi = jax.lax.Precision.HIGHEST
    pool = _region_onehot((NREG, N)) * (1.0 / 9.0)
    xp = jnp.dot(pool, x1, precision=hi, preferred_element_type=jnp.float32)
    wt = wt_ref[...]
    nt = (((1,), (1,)), ((), ()))
    q_r = jnp.dot(xp, wt[:, :C], precision=hi,
                  preferred_element_type=jnp.float32) + wb_ref[0, :C]
    k_r = jnp.dot(xp, wt[:, C:2 * C], precision=hi,
                  preferred_element_type=jnp.float32) + wb_ref[0, C:2 * C]
    a = jax.lax.dot_general(q_r, k_r, nt,
                            precision=hi, preferred_element_type=jnp.float32)
    m_ref[0] = a


_NW = 32
_ROWS = 8 * NREG
_RPW = _ROWS // _NW


def _route_sc_kernel(a_hbm, m_hbm, a_v, m_v):
    c = jax.lax.axis_index("c")
    s = jax.lax.axis_index("s")
    wid = s * 2 + c
    base = wid * _RPW
    pltpu.sync_copy(a_hbm.at[pl.ds(base, _RPW)], a_v)
    lane = jax.lax.iota(jnp.int32, 16)
    for r in range(_RPW):
        work = [a_v[r, 16 * j:16 * (j + 1)] for j in range(4)]
        sel = [jnp.zeros((16,), jnp.bool_) for _ in range(4)]
        for _ in range(TK):
            mx = jnp.max(jnp.maximum(jnp.maximum(work[0], work[1]),
                                     jnp.maximum(work[2], work[3])))
            pos = [jnp.min(jnp.where(work[j] == mx, lane + 16 * j, NREG))
                   for j in range(4)]
            first = jnp.minimum(jnp.minimum(pos[0], pos[1]),
                                jnp.minimum(pos[2], pos[3]))
            for j in range(4):
                hit = (lane + 16 * j) == first
                sel[j] = jnp.logical_or(sel[j], hit)
                work[j] = jnp.where(hit, -jnp.inf, work[j])
        for j in range(4):
            m_v[r, 16 * j:16 * (j + 1)] = jnp.where(sel[j], 0.0, NEG)
    pltpu.sync_copy(m_v, m_hbm.at[pl.ds(base, _RPW)])


@functools.partial(
    pl.kernel,
    mesh=plsc.VectorSubcoreMesh(core_axis_name="c", subcore_axis_name="s"),
    out_type=jax.ShapeDtypeStruct((_ROWS, NREG), jnp.float32),
    scratch_types=[
        pltpu.VMEM((_RPW, NREG), jnp.float32),
        pltpu.VMEM((_RPW, NREG), jnp.float32),
    ],
    compiler_params=pltpu.CompilerParams(needs_layout_passes=False),
)
def _route_sc(a_hbm, m_hbm, a_v, m_v):
    _route_sc_kernel(a_hbm, m_hbm, a_v, m_v)


def _block_kernel(x_ref, q_ref, k_ref, v_ref, m_ref, lw_ref, lb_ref,
                  ot_ref, ob_ref, g2_ref, b2_ref, f1_ref, f1b_ref,
                  f2_ref, f2b_ref, y_ref):
    x = x_ref[0]
    q = q_ref[0]
    k = k_ref[0]
    v = v_ref[0]

    e_rn = _region_onehot((NREG, N))
    m64 = m_ref[0]
    inner = jnp.dot(m64, e_rn, preferred_element_type=jnp.float32)
    mask = jax.lax.dot_general(e_rn, inner, (((0,), (0,)), ((), ())),
                               preferred_element_type=jnp.float32)

    wcol = jax.lax.broadcasted_iota(jnp.int32, (N, 1), 0) % RROW
    zrow = jnp.zeros((RROW + 1, C), jnp.bfloat16)
    lepe = jnp.zeros((N, C), jnp.float32) + lb_ref[0]
    for kh in range(3):
        for kw in range(3):
            s = RROW * (kh - 1) + (kw - 1)
            if s > 0:
                sh = jnp.concatenate([v[s:], zrow[:s]], axis=0)
            elif s < 0:
                sh = jnp.concatenate([zrow[:-s], v[:N + s]], axis=0)
            else:
                sh = v
            if kw == 0:
                sh = jnp.where(wcol >= 1, sh, jnp.bfloat16(0))
            elif kw == 2:
                sh = jnp.where(wcol <= RROW - 2, sh, jnp.bfloat16(0))
            lepe = lepe + sh * lw_ref[kh * 3 + kw][None, :].astype(jnp.bfloat16)

    ones_bf = jnp.ones((N, 128), jnp.bfloat16)
    mask_bf = mask.astype(jnp.bfloat16)
    heads = []
    for h in range(NH):
        sl = slice(h * HD, (h + 1) * HD)
        s = jax.lax.dot_general(q[:, sl], k[:, sl], (((1,), (1,)), ((), ())),
                                preferred_element_type=jnp.float32)
        e = jnp.exp(s.astype(jnp.bfloat16) + mask_bf)
        denom = jnp.dot(e, ones_bf, preferred_element_type=jnp.float32)
        o = jnp.dot(e, v[:, sl], preferred_element_type=jnp.float32)
        heads.append(o / denom[:, :HD])

    proj = jnp.zeros((N, C), jnp.float32)
    for hp in range(NH // 2):
        sl = slice(hp * 128, (hp + 1) * 128)
        ab = (jnp.concatenate([heads[2 * hp], heads[2 * hp + 1]], axis=1)
              + lepe[:, sl]).astype(jnp.bfloat16)
        proj = proj + jax.lax.dot_general(
            ab, ot_ref[:, sl], (((1,), (1,)), ((), ())),
            preferred_element_type=jnp.float32)
    xm = x + proj + ob_ref[0]

    x2 = _layer_norm_f32(xm, g2_ref[0], b2_ref[0]).astype(jnp.bfloat16)
    yacc = jnp.zeros((N, C), jnp.float32)
    chunk = MLP_H // 4
    for j in range(4):
        sl = slice(j * chunk, (j + 1) * chunk)
        h1 = jax.lax.dot_general(
            x2, f1_ref[sl, :], (((1,), (1,)), ((), ())),
            preferred_element_type=jnp.float32) + f1b_ref[0, sl]
        gl = 0.5 * h1 * (1.0 + jax.lax.erf(h1 * (2.0 ** -0.5)))
        yacc = yacc + jax.lax.dot_general(
            gl.astype(jnp.bfloat16), f2_ref[:, sl], (((1,), (1,)), ((), ())),
            preferred_element_type=jnp.float32)
    y_ref[0] = xm + yacc + f2b_ref[0]


def _full(shape):
    return pl.BlockSpec(shape, lambda b: (0,) * len(shape))


def _batched(shape):
    return pl.BlockSpec((1,) + shape, lambda b: (b,) + (0,) * len(shape))


@jax.jit
def kernel(x, norm1_g, norm1_b, qkv_w, qkv_b, lepe_w, lepe_b, out_w, out_b,
           norm2_g, norm2_b, fc1_w, fc1_b, fc2_w, fc2_b):
    B = x.shape[0]
    f32 = jnp.float32
    bf16 = jnp.bfloat16

    q, k, v, a_r = pl.pallas_call(
        _qkv_kernel,
        grid=(B,),
        in_specs=[
            _batched((N, C)),
            _full((1, C)), _full((1, C)),
            _full((C, 3 * C)), _full((1, 3 * C)),
        ],
        out_specs=[
            _batched((N, C)), _batched((N, C)), _batched((N, C)),
            _batched((NREG, NREG)),
        ],
        out_shape=[
            jax.ShapeDtypeStruct((B, N, C), bf16),
            jax.ShapeDtypeStruct((B, N, C), bf16),
            jax.ShapeDtypeStruct((B, N, C), bf16),
            jax.ShapeDtypeStruct((B, NREG, NREG), f32),
        ],
    )(x, norm1_g.reshape(1, C), norm1_b.reshape(1, C),
      qkv_w.T, qkv_b.reshape(1, 3 * C))

    mask64 = _route_sc(a_r.reshape(_ROWS, NREG)).reshape(B, NREG, NREG)

    lw9 = jnp.transpose(lepe_w, (1, 2, 3, 0)).reshape(9, C)
    y = pl.pallas_call(
        _block_kernel,
        grid=(B,),
        in_specs=[
            _batched((N, C)), _batched((N, C)), _batched((N, C)),
            _batched((N, C)), _batched((NREG, NREG)),
            _full((9, C)), _full((1, C)),
            _full((C, C)), _full((1, C)),
            _full((1, C)), _full((1, C)),
            _full((MLP_H, C)), _full((1, MLP_H)),
            _full((C, MLP_H)), _full((1, C)),
        ],
        out_specs=_batched((N, C)),
        out_shape=jax.ShapeDtypeStruct((B, N, C), f32),
    )(x, q, k, v, mask64,
      lw9, lepe_b.reshape(1, C),
      out_w.astype(bf16), out_b.reshape(1, C),
      norm2_g.reshape(1, C), norm2_b.reshape(1, C),
      fc1_w.astype(bf16), fc1_b.reshape(1, MLP_H),
      fc2_w.astype(bf16), fc2_b.reshape(1, C))
    return y


# speedup vs baseline: 1.0655x; 1.0210x over previous
import functools

import jax
import jax.numpy as jnp
from jax.experimental import pallas as pl
from jax.experimental.pallas import tpu as pltpu
from jax.experimental.pallas import tpu_sc as plsc

N = 576
C = 768
NH = 12
HD = 64
NREG = 64
RROW = 24
TK = 4
MLP_H = 3072
NEG = -1e30


def _layer_norm_f32(x, g, b):
    m = jnp.mean(x, axis=-1, keepdims=True)
    v = jnp.mean((x - m) ** 2, axis=-1, keepdims=True)
    return (x - m) * jax.lax.rsqrt(v + 1e-5) * g + b


def _region_onehot(shape_rn):
    r_idx = jax.lax.broadcasted_iota(jnp.int32, shape_rn, 0)
    n_idx = jax.lax.broadcasted_iota(jnp.int32, shape_rn, 1)
    rid = (n_idx // 72) * 8 + (n_idx % RROW) // 3
    return jnp.where(rid == r_idx, 1.0, 0.0).astype(jnp.float32)


def _qkv_kernel(x_ref, g_ref, b_ref, wt_ref, wb_ref,
                q_ref, k_ref, v_ref, m_ref):
    x = x_ref[0]
    g = g_ref[0]
    b = b_ref[0]
    x1 = _layer_norm_f32(x, g, b)

    wt_bf = wt_ref[...].astype(jnp.bfloat16)
    qkv = jnp.dot(x1.astype(jnp.bfloat16), wt_bf,
                  preferred_element_type=jnp.float32) + wb_ref[0]
    q_ref[0] = (qkv[:, :C] * (float(C) ** -0.5)).astype(jnp.bfloat16)
    k_ref[0] = qkv[:, C:2 * C].astype(jnp.bfloat16)
    v_ref[0] = qkv[:, 2 * C:].astype(jnp.bfloat16)

    hi = jax.lax.Precision.HIGHEST
    pool = _region_onehot((NREG, N)) * (1.0 / 9.0)
    xp = jnp.dot(pool, x1, precision=hi, preferred_element_type=jnp.float32)
    wt = wt_ref[...]
    nt = (((1,), (1,)), ((), ()))
    q_r = jnp.dot(xp, wt[:, :C], precision=hi,
                  preferred_element_type=jnp.float32) + wb_ref[0, :C]
    k_r = jnp.dot(xp, wt[:, C:2 * C], precision=hi,
                  preferred_element_type=jnp.float32) + wb_ref[0, C:2 * C]
    a = jax.lax.dot_general(q_r, k_r, nt,
                            precision=hi, preferred_element_type=jnp.float32)
    m_ref[0] = a


_NW = 32
_ROWS = 8 * NREG
_RPW = _ROWS // _NW


def _route_sc_kernel(a_hbm, m_hbm, a_v, m_v):
    c = jax.lax.axis_index("c")
    s = jax.lax.axis_index("s")
    wid = s * 2 + c
    base = wid * _RPW
    pltpu.sync_copy(a_hbm.at[pl.ds(base, _RPW)], a_v)
    lane = jax.lax.iota(jnp.int32, 16)
    for r in range(_RPW):
        work = [a_v[r, 16 * j:16 * (j + 1)] for j in range(4)]
        sel = [jnp.zeros((16,), jnp.bool_) for _ in range(4)]
        for _ in range(TK):
            mx = jnp.max(jnp.maximum(jnp.maximum(work[0], work[1]),
                                     jnp.maximum(work[2], work[3])))
            pos = [jnp.min(jnp.where(work[j] == mx, lane + 16 * j, NREG))
                   for j in range(4)]
            first = jnp.minimum(jnp.minimum(pos[0], pos[1]),
                                jnp.minimum(pos[2], pos[3]))
            for j in range(4):
                hit = (lane + 16 * j) == first
                sel[j] = jnp.logical_or(sel[j], hit)
                work[j] = jnp.where(hit, -jnp.inf, work[j])
        for j in range(4):
            m_v[r, 16 * j:16 * (j + 1)] = jnp.where(sel[j], 0.0, NEG)
    pltpu.sync_copy(m_v, m_hbm.at[pl.ds(base, _RPW)])


@functools.partial(
    pl.kernel,
    mesh=plsc.VectorSubcoreMesh(core_axis_name="c", subcore_axis_name="s"),
    out_type=jax.ShapeDtypeStruct((_ROWS, NREG), jnp.float32),
    scratch_types=[
        pltpu.VMEM((_RPW, NREG), jnp.float32),
        pltpu.VMEM((_RPW, NREG), jnp.float32),
    ],
    compiler_params=pltpu.CompilerParams(needs_layout_passes=False),
)
def _route_sc(a_hbm, m_hbm, a_v, m_v):
    _route_sc_kernel(a_hbm, m_hbm, a_v, m_v)


def _block_kernel(x_ref, q_ref, k_ref, v_ref, m_ref, lw_ref, lb_ref,
                  ot_ref, ob_ref, g2_ref, b2_ref, f1_ref, f1b_ref,
                  f2_ref, f2b_ref, y_ref):
    x = x_ref[0]
    q = q_ref[0]
    k = k_ref[0]
    v = v_ref[0]

    e_rn = _region_onehot((NREG, N)).astype(jnp.bfloat16)
    m64 = m_ref[0].astype(jnp.bfloat16)
    inner = jnp.dot(m64, e_rn,
                    preferred_element_type=jnp.float32).astype(jnp.bfloat16)
    mask_bf = jax.lax.dot_general(
        e_rn, inner, (((0,), (0,)), ((), ())),
        preferred_element_type=jnp.float32).astype(jnp.bfloat16)

    wcol = jax.lax.broadcasted_iota(jnp.int32, (N, 1), 0) % RROW
    zrow = jnp.zeros((RROW + 1, C), jnp.bfloat16)
    lwb = lw_ref[...].astype(jnp.bfloat16)
    lepe = jnp.zeros((N, C), jnp.bfloat16) + lb_ref[0].astype(jnp.bfloat16)
    for kh in range(3):
        for kw in range(3):
            s = RROW * (kh - 1) + (kw - 1)
            if s > 0:
                sh = jnp.concatenate([v[s:], zrow[:s]], axis=0)
            elif s < 0:
                sh = jnp.concatenate([zrow[:-s], v[:N + s]], axis=0)
            else:
                sh = v
            if kw == 0:
                sh = jnp.where(wcol >= 1, sh, jnp.bfloat16(0))
            elif kw == 2:
                sh = jnp.where(wcol <= RROW - 2, sh, jnp.bfloat16(0))
            lepe = lepe + sh * lwb[kh * 3 + kw][None, :]

    ones_bf = jnp.ones((N, 128), jnp.bfloat16)
    heads = []
    for h in range(NH):
        sl = slice(h * HD, (h + 1) * HD)
        s = jax.lax.dot_general(q[:, sl], k[:, sl], (((1,), (1,)), ((), ())),
                                preferred_element_type=jnp.float32)
        e = jnp.exp(s.astype(jnp.bfloat16) + mask_bf)
        denom = jnp.dot(e, ones_bf, preferred_element_type=jnp.float32)
        o = jnp.dot(e, v[:, sl], preferred_element_type=jnp.float32)
        heads.append(o / denom[:, :HD])

    proj = jnp.zeros((N, C), jnp.float32)
    for hp in range(NH // 2):
        sl = slice(hp * 128, (hp + 1) * 128)
        ab = (jnp.concatenate([heads[2 * hp], heads[2 * hp + 1]], axis=1)
              + lepe[:, sl]).astype(jnp.bfloat16)
        proj = proj + jax.lax.dot_general(
            ab, ot_ref[:, sl], (((1,), (1,)), ((), ())),
            preferred_element_type=jnp.float32)
    xm = x + proj + ob_ref[0]

    x2 = _layer_norm_f32(xm, g2_ref[0], b2_ref[0]).astype(jnp.bfloat16)
    yacc = jnp.zeros((N, C), jnp.float32)
    chunk = MLP_H // 4
    for j in range(4):
        sl = slice(j * chunk, (j + 1) * chunk)
        h1 = jax.lax.dot_general(
            x2, f1_ref[sl, :], (((1,), (1,)), ((), ())),
            preferred_element_type=jnp.float32) + f1b_ref[0, sl]
        gl = 0.5 * h1 * (1.0 + jax.lax.erf(h1 * (2.0 ** -0.5)))
        yacc = yacc + jax.lax.dot_general(
            gl.astype(jnp.bfloat16), f2_ref[:, sl], (((1,), (1,)), ((), ())),
            preferred_element_type=jnp.float32)
    y_ref[0] = xm + yacc + f2b_ref[0]


def _full(shape):
    return pl.BlockSpec(shape, lambda b: (0,) * len(shape))


def _batched(shape):
    return pl.BlockSpec((1,) + shape, lambda b: (b,) + (0,) * len(shape))


@jax.jit
def kernel(x, norm1_g, norm1_b, qkv_w, qkv_b, lepe_w, lepe_b, out_w, out_b,
           norm2_g, norm2_b, fc1_w, fc1_b, fc2_w, fc2_b):
    B = x.shape[0]
    f32 = jnp.float32
    bf16 = jnp.bfloat16

    q, k, v, a_r = pl.pallas_call(
        _qkv_kernel,
        grid=(B,),
        in_specs=[
            _batched((N, C)),
            _full((1, C)), _full((1, C)),
            _full((C, 3 * C)), _full((1, 3 * C)),
        ],
        out_specs=[
            _batched((N, C)), _batched((N, C)), _batched((N, C)),
            _batched((NREG, NREG)),
        ],
        out_shape=[
            jax.ShapeDtypeStruct((B, N, C), bf16),
            jax.ShapeDtypeStruct((B, N, C), bf16),
            jax.ShapeDtypeStruct((B, N, C), bf16),
            jax.ShapeDtypeStruct((B, NREG, NREG), f32),
        ],
    )(x, norm1_g.reshape(1, C), norm1_b.reshape(1, C),
      qkv_w.T, qkv_b.reshape(1, 3 * C))

    mask64 = _route_sc(a_r.reshape(_ROWS, NREG)).reshape(B, NREG, NREG)

    lw9 = jnp.transpose(lepe_w, (1, 2, 3, 0)).reshape(9, C)
    y = pl.pallas_call(
        _block_kernel,
        grid=(B,),
        in_specs=[
            _batched((N, C)), _batched((N, C)), _batched((N, C)),
            _batched((N, C)), _batched((NREG, NREG)),
            _full((9, C)), _full((1, C)),
            _full((C, C)), _full((1, C)),
            _full((1, C)), _full((1, C)),
            _full((MLP_H, C)), _full((1, MLP_H)),
            _full((C, MLP_H)), _full((1, C)),
        ],
        out_specs=_batched((N, C)),
        out_shape=jax.ShapeDtypeStruct((B, N, C), f32),
    )(x, q, k, v, mask64,
      lw9, lepe_b.reshape(1, C),
      out_w.astype(bf16), out_b.reshape(1, C),
      norm2_g.reshape(1, C), norm2_b.reshape(1, C),
      fc1_w.astype(bf16), fc1_b.reshape(1, MLP_H),
      fc2_w.astype(bf16), fc2_b.reshape(1, C))
    return y


# speedup vs baseline: 1.1690x; 1.0972x over previous
import functools

import jax
import jax.numpy as jnp
from jax.experimental import pallas as pl
from jax.experimental.pallas import tpu as pltpu
from jax.experimental.pallas import tpu_sc as plsc

N = 576
C = 768
NH = 12
HD = 64
NREG = 64
RROW = 24
TK = 4
MLP_H = 3072
NEG = -1e30


def _layer_norm_f32(x, g, b):
    m = jnp.mean(x, axis=-1, keepdims=True)
    v = jnp.mean((x - m) ** 2, axis=-1, keepdims=True)
    return (x - m) * jax.lax.rsqrt(v + 1e-5) * g + b


def _region_onehot(shape_rn):
    r_idx = jax.lax.broadcasted_iota(jnp.int32, shape_rn, 0)
    n_idx = jax.lax.broadcasted_iota(jnp.int32, shape_rn, 1)
    rid = (n_idx // 72) * 8 + (n_idx % RROW) // 3
    return jnp.where(rid == r_idx, 1.0, 0.0).astype(jnp.float32)


def _qkv_kernel(x_ref, g_ref, b_ref, wt_ref, wb_ref,
                q_ref, k_ref, v_ref, m_ref):
    x = x_ref[0]
    g = g_ref[0]
    b = b_ref[0]
    x1 = _layer_norm_f32(x, g, b)

    wt_bf = wt_ref[...].astype(jnp.bfloat16)
    qkv = jnp.dot(x1.astype(jnp.bfloat16), wt_bf,
                  preferred_element_type=jnp.float32) + wb_ref[0]
    q_ref[0] = (qkv[:, :C] * (float(C) ** -0.5)).astype(jnp.bfloat16)
    k_ref[0] = qkv[:, C:2 * C].astype(jnp.bfloat16)
    v_ref[0] = qkv[:, 2 * C:].astype(jnp.bfloat16)

    hi = jax.lax.Precision.HIGHEST
    pool = _region_onehot((NREG, N)) * (1.0 / 9.0)
    xp = jnp.dot(pool, x1, precision=hi, preferred_element_type=jnp.float32)
    wt = wt_ref[...]
    nt = (((1,), (1,)), ((), ()))
    q_r = jnp.dot(xp, wt[:, :C], precision=hi,
                  preferred_element_type=jnp.float32) + wb_ref[0, :C]
    k_r = jnp.dot(xp, wt[:, C:2 * C], precision=hi,
                  preferred_element_type=jnp.float32) + wb_ref[0, C:2 * C]
    a = jax.lax.dot_general(q_r, k_r, nt,
                            precision=hi, preferred_element_type=jnp.float32)
    m_ref[0] = a


_NW = 32
_ROWS = 8 * NREG
_RPW = _ROWS // _NW


def _route_sc_kernel(a_hbm, m_hbm, a_v, m_v):
    c = jax.lax.axis_index("c")
    s = jax.lax.axis_index("s")
    wid = s * 2 + c
    base = wid * _RPW
    pltpu.sync_copy(a_hbm.at[pl.ds(base, _RPW)], a_v)
    lane = jax.lax.iota(jnp.int32, 16)
    for r in range(_RPW):
        work = [a_v[r, 16 * j:16 * (j + 1)] for j in range(4)]
        sel = [jnp.zeros((16,), jnp.bool_) for _ in range(4)]
        for _ in range(TK):
            mx = jnp.max(jnp.maximum(jnp.maximum(work[0], work[1]),
                                     jnp.maximum(work[2], work[3])))
            pos = [jnp.min(jnp.where(work[j] == mx, lane + 16 * j, NREG))
                   for j in range(4)]
            first = jnp.minimum(jnp.minimum(pos[0], pos[1]),
                                jnp.minimum(pos[2], pos[3]))
            for j in range(4):
                hit = (lane + 16 * j) == first
                sel[j] = jnp.logical_or(sel[j], hit)
                work[j] = jnp.where(hit, -jnp.inf, work[j])
        for j in range(4):
            m_v[r, 16 * j:16 * (j + 1)] = jnp.where(sel[j], 0.0, NEG)
    pltpu.sync_copy(m_v, m_hbm.at[pl.ds(base, _RPW)])


@functools.partial(
    pl.kernel,
    mesh=plsc.VectorSubcoreMesh(core_axis_name="c", subcore_axis_name="s"),
    out_type=jax.ShapeDtypeStruct((_ROWS, NREG), jnp.float32),
    scratch_types=[
        pltpu.VMEM((_RPW, NREG), jnp.float32),
        pltpu.VMEM((_RPW, NREG), jnp.float32),
    ],
    compiler_params=pltpu.CompilerParams(needs_layout_passes=False),
)
def _route_sc(a_hbm, m_hbm, a_v, m_v):
    _route_sc_kernel(a_hbm, m_hbm, a_v, m_v)


def _block_kernel(x_ref, q_ref, k_ref, v_ref, m_ref, lw_ref, lb_ref,
                  ot_ref, ob_ref, g2_ref, b2_ref, f1_ref, f1b_ref,
                  f2_ref, f2b_ref, y_ref):
    x = x_ref[0]
    q = q_ref[0]
    k = k_ref[0]
    v = v_ref[0]

    e_rn = _region_onehot((NREG, N)).astype(jnp.bfloat16)
    m64 = m_ref[0].astype(jnp.bfloat16)
    inner = jnp.dot(m64, e_rn,
                    preferred_element_type=jnp.float32).astype(jnp.bfloat16)
    mask_bf = jax.lax.dot_general(
        e_rn, inner, (((0,), (0,)), ((), ())),
        preferred_element_type=jnp.float32).astype(jnp.bfloat16)

    wcol = jax.lax.broadcasted_iota(jnp.int32, (N, 1), 0) % RROW
    zrow = jnp.zeros((RROW + 1, C), jnp.bfloat16)
    lwb = lw_ref[...].astype(jnp.bfloat16)
    lepe = jnp.zeros((N, C), jnp.bfloat16) + lb_ref[0].astype(jnp.bfloat16)
    for kh in range(3):
        for kw in range(3):
            s = RROW * (kh - 1) + (kw - 1)
            if s > 0:
                sh = jnp.concatenate([v[s:], zrow[:s]], axis=0)
            elif s < 0:
                sh = jnp.concatenate([zrow[:-s], v[:N + s]], axis=0)
            else:
                sh = v
            if kw == 0:
                sh = jnp.where(wcol >= 1, sh, jnp.bfloat16(0))
            elif kw == 2:
                sh = jnp.where(wcol <= RROW - 2, sh, jnp.bfloat16(0))
            lepe = lepe + sh * lwb[kh * 3 + kw][None, :]

    ones_bf = jnp.ones((N, HD), jnp.bfloat16)
    heads = []
    for h in range(NH):
        sl = slice(h * HD, (h + 1) * HD)
        s = jax.lax.dot_general(q[:, sl], k[:, sl], (((1,), (1,)), ((), ())),
                                preferred_element_type=jnp.float32)
        e = jnp.exp(s.astype(jnp.bfloat16) + mask_bf)
        vx = jnp.concatenate([v[:, sl], ones_bf], axis=1)
        o = jnp.dot(e, vx, preferred_element_type=jnp.float32)
        heads.append(o[:, :HD] / o[:, HD:HD + 1])

    proj = jnp.zeros((N, C), jnp.float32)
    for hp in range(NH // 2):
        sl = slice(hp * 128, (hp + 1) * 128)
        ab = (jnp.concatenate([heads[2 * hp], heads[2 * hp + 1]], axis=1)
              + lepe[:, sl]).astype(jnp.bfloat16)
        proj = proj + jax.lax.dot_general(
            ab, ot_ref[:, sl], (((1,), (1,)), ((), ())),
            preferred_element_type=jnp.float32)
    xm = x + proj + ob_ref[0]

    x2 = _layer_norm_f32(xm, g2_ref[0], b2_ref[0]).astype(jnp.bfloat16)
    yacc = jnp.zeros((N, C), jnp.float32)
    chunk = MLP_H // 4
    for j in range(4):
        sl = slice(j * chunk, (j + 1) * chunk)
        h1 = jax.lax.dot_general(
            x2, f1_ref[sl, :], (((1,), (1,)), ((), ())),
            preferred_element_type=jnp.float32) + f1b_ref[0, sl]
        hb = h1.astype(jnp.bfloat16)
        gl = jnp.bfloat16(0.5) * hb * (jnp.bfloat16(1.0)
             + jax.lax.erf(hb * jnp.bfloat16(2.0 ** -0.5)))
        yacc = yacc + jax.lax.dot_general(
            gl, f2_ref[:, sl], (((1,), (1,)), ((), ())),
            preferred_element_type=jnp.float32)
    y_ref[0] = xm + yacc + f2b_ref[0]


def _full(shape):
    return pl.BlockSpec(shape, lambda b: (0,) * len(shape))


def _batched(shape):
    return pl.BlockSpec((1,) + shape, lambda b: (b,) + (0,) * len(shape))


@jax.jit
def kernel(x, norm1_g, norm1_b, qkv_w, qkv_b, lepe_w, lepe_b, out_w, out_b,
           norm2_g, norm2_b, fc1_w, fc1_b, fc2_w, fc2_b):
    B = x.shape[0]
    f32 = jnp.float32
    bf16 = jnp.bfloat16

    q, k, v, a_r = pl.pallas_call(
        _qkv_kernel,
        grid=(B,),
        in_specs=[
            _batched((N, C)),
            _full((1, C)), _full((1, C)),
            _full((C, 3 * C)), _full((1, 3 * C)),
        ],
        out_specs=[
            _batched((N, C)), _batched((N, C)), _batched((N, C)),
            _batched((NREG, NREG)),
        ],
        out_shape=[
            jax.ShapeDtypeStruct((B, N, C), bf16),
            jax.ShapeDtypeStruct((B, N, C), bf16),
            jax.ShapeDtypeStruct((B, N, C), bf16),
            jax.ShapeDtypeStruct((B, NREG, NREG), f32),
        ],
    )(x, norm1_g.reshape(1, C), norm1_b.reshape(1, C),
      qkv_w.T, qkv_b.reshape(1, 3 * C))

    mask64 = _route_sc(a_r.reshape(_ROWS, NREG)).reshape(B, NREG, NREG)

    lw9 = jnp.transpose(lepe_w, (1, 2, 3, 0)).reshape(9, C)
    y = pl.pallas_call(
        _block_kernel,
        grid=(B,),
        in_specs=[
            _batched((N, C)), _batched((N, C)), _batched((N, C)),
            _batched((N, C)), _batched((NREG, NREG)),
            _full((9, C)), _full((1, C)),
            _full((C, C)), _full((1, C)),
            _full((1, C)), _full((1, C)),
            _full((MLP_H, C)), _full((1, MLP_H)),
            _full((C, MLP_H)), _full((1, C)),
        ],
        out_specs=_batched((N, C)),
        out_shape=jax.ShapeDtypeStruct((B, N, C), f32),
    )(x, q, k, v, mask64,
      lw9, lepe_b.reshape(1, C),
      out_w.astype(bf16), out_b.reshape(1, C),
      norm2_g.reshape(1, C), norm2_b.reshape(1, C),
      fc1_w.astype(bf16), fc1_b.reshape(1, MLP_H),
      fc2_w.astype(bf16), fc2_b.reshape(1, C))
    return y
